# baseline (speedup 1.0000x reference)
import jax
import jax.numpy as jnp
from jax.experimental import pallas as pl
from jax.experimental.pallas import tpu as pltpu

L = 8
FFT = 256
NUM_CLS = 256
COND = 80
HOP = 64
FRAMES = 4
T = FRAMES * HOP
DILATIONS = [128, 64, 32, 16, 8, 4, 2, 1]


def _prefix_sum(p):
    lane = jax.lax.broadcasted_iota(jnp.int32, (1, NUM_CLS), 1)
    c = p
    sh = 1
    while sh < NUM_CLS:
        c = c + jnp.where(lane >= sh, jnp.roll(c, sh, axis=1), 0.0)
        sh *= 2
    return c


def _body(yup_ref, s_ref, emb_ref, cw_ref, wp_ref, wpr_ref, wo_ref, wob_ref,
          ew_ref, eb_ref, out_ref,
          h1, h2, h3, h4, h5, h6, h7, conds, m0p, m0r, idxh):
    conds[:, :] = jnp.dot(yup_ref[:, :], cw_ref[:, :],
                          preferred_element_type=jnp.float32)
    m0p[:, :] = jnp.dot(emb_ref[:, :], wp_ref[0],
                        preferred_element_type=jnp.float32)
    m0r[:, :] = jnp.dot(emb_ref[:, :], wpr_ref[0],
                        preferred_element_type=jnp.float32)
    idxh[0, 0] = NUM_CLS // 2 - 1
    hists = [h1, h2, h3, h4, h5, h6, h7]
    zero_row = jnp.zeros((1, FFT), jnp.float32)

    def step(t, carry):
        k = carry[0]
        pcs = carry[1:]

        kp = idxh[jnp.maximum(t - DILATIONS[0], 0), 0]
        past0 = jnp.where(t >= DILATIONS[0], m0p[pl.ds(kp, 1), :], 0.0)
        h = conds[pl.ds(t, 1), 0:FFT] + past0
        h = h + m0r[pl.ds(k, 1), :]
        h = jnp.maximum(h, 0.0)
        z = wob_ref[0:1, :] + jnp.dot(h, wo_ref[0],
                                      preferred_element_type=jnp.float32)
        x = jnp.maximum(z + emb_ref[pl.ds(k, 1), :], 0.0)

        x7 = None
        for j in range(1, L):
            hj = hists[j - 1]
            h = conds[pl.ds(t, 1), j * FFT:(j + 1) * FFT] + pcs[j - 1]
            h = h + jnp.dot(x, wpr_ref[j], preferred_element_type=jnp.float32)
            hj[pl.ds(t, 1), :] = x
            if j == L - 1:
                x7 = x
            h = jnp.maximum(h, 0.0)
            z = wob_ref[j:j + 1, :] + jnp.dot(h, wo_ref[j],
                                              preferred_element_type=jnp.float32)
            x = jnp.maximum(z + x, 0.0)

        new_pcs = []
        for j in range(1, L):
            d = DILATIONS[j]
            if d == 1:
                xp = x7
                new_pcs.append(jnp.dot(xp, wp_ref[j],
                                       preferred_element_type=jnp.float32))
            else:
                tp = t + 1 - d
                xp = hists[j - 1][pl.ds(jnp.maximum(tp, 0), 1), :]
                pc = jnp.dot(xp, wp_ref[j], preferred_element_type=jnp.float32)
                new_pcs.append(jnp.where(tp >= 0, pc, 0.0))

        logits = eb_ref[:, :] + jnp.dot(x, ew_ref[:, :],
                                        preferred_element_type=jnp.float32)
        m = jnp.max(logits)
        e = jnp.exp(logits - m)
        p = e / jnp.sum(e)
        cum = _prefix_sum(p)
        s = s_ref[pl.ds(t, 1), :]
        cnt = jnp.sum((cum <= s).astype(jnp.int32))
        nx = jnp.bitwise_and(cnt, NUM_CLS - 1)
        out_ref[pl.ds(t, 1), :] = nx[None, None]
        idxh[t + 1, 0] = nx
        return (nx,) + tuple(new_pcs)

    carry0 = (jnp.int32(NUM_CLS // 2 - 1),) + tuple(zero_row for _ in range(L - 1))
    jax.lax.fori_loop(0, T, step, carry0)


def kernel(y, samples, emb, condition_W, WV_past_weight, WV_present_weight,
           W_o_weight, W_o_bias, end_w, end_b):
    y_up_t = jnp.repeat(y, HOP, axis=1).T
    cw_t = condition_W.T
    wp_t = WV_past_weight[:, :, :, 0].transpose(0, 2, 1)
    wpr_t = WV_present_weight.transpose(0, 2, 1)
    wo_t = W_o_weight.transpose(0, 2, 1)
    ew_t = end_w.T
    s2 = samples.reshape(T, 1)
    eb2 = end_b.reshape(1, NUM_CLS)

    hist_scratch = [pltpu.VMEM((T, FFT), jnp.float32) for _ in range(L - 1)]
    out = pl.pallas_call(
        _body,
        out_shape=jax.ShapeDtypeStruct((T, 1), jnp.int32),
        scratch_shapes=hist_scratch + [
            pltpu.VMEM((T, L * FFT), jnp.float32),
            pltpu.VMEM((NUM_CLS, FFT), jnp.float32),
            pltpu.VMEM((NUM_CLS, FFT), jnp.float32),
            pltpu.SMEM((T + 1, 1), jnp.int32),
        ],
    )(y_up_t, s2, emb, cw_t, wp_t, wpr_t, wo_t, W_o_bias, ew_t, eb2)
    return out[:, 0]


# speedup vs baseline: 1.0970x; 1.0970x over previous
import jax
import jax.numpy as jnp
from jax.experimental import pallas as pl
from jax.experimental.pallas import tpu as pltpu

L = 8
FFT = 256
NUM_CLS = 256
COND = 80
HOP = 64
FRAMES = 4
T = FRAMES * HOP
DILATIONS = [128, 64, 32, 16, 8, 4, 2, 1]


def _body(yup_ref, s_ref, emb_ref, cw_ref, wp_ref, wpr_ref, wo_ref, wob_ref,
          ew_ref, eb_ref, tri_ref, out_ref,
          h1, h2, h3, h4, h5, h6, h7, conds, m0p, m0r, idxh):
    conds[:, :] = jnp.dot(yup_ref[:, :], cw_ref[:, :],
                          preferred_element_type=jnp.float32)
    m0p[0:NUM_CLS, :] = jnp.dot(emb_ref[:, :], wp_ref[0],
                                preferred_element_type=jnp.float32)
    m0p[NUM_CLS:NUM_CLS + 8, :] = jnp.zeros((8, FFT), jnp.float32)
    m0r[:, :] = jnp.dot(emb_ref[:, :], wpr_ref[0],
                        preferred_element_type=jnp.float32)
    hists = [h1, h2, h3, h4, h5, h6, h7]
    for j in range(1, L):
        d = DILATIONS[j]
        hists[j - 1][0:d, :] = jnp.zeros((d, FFT), jnp.float32)

    def init_idx(u, c):
        idxh[u, 0] = NUM_CLS
        return c
    jax.lax.fori_loop(0, DILATIONS[0], init_idx, 0)
    idxh[DILATIONS[0], 0] = NUM_CLS // 2 - 1

    def step(t, k):
        kp = idxh[t, 0]
        h = conds[pl.ds(t, 1), 0:FFT] + m0p[pl.ds(kp, 1), :]
        h = h + m0r[pl.ds(k, 1), :]
        h = jnp.maximum(h, 0.0)
        z = wob_ref[0:1, :] + jnp.dot(h, wo_ref[0],
                                      preferred_element_type=jnp.float32)
        x = jnp.maximum(z + emb_ref[pl.ds(k, 1), :], 0.0)

        for j in range(1, L):
            d = DILATIONS[j]
            hj = hists[j - 1]
            xpast = hj[pl.ds(t, 1), :]
            h = conds[pl.ds(t, 1), j * FFT:(j + 1) * FFT]
            h = h + jnp.dot(xpast, wp_ref[j], preferred_element_type=jnp.float32)
            h = h + jnp.dot(x, wpr_ref[j], preferred_element_type=jnp.float32)
            hj[pl.ds(t + d, 1), :] = x
            h = jnp.maximum(h, 0.0)
            z = wob_ref[j:j + 1, :] + jnp.dot(h, wo_ref[j],
                                              preferred_element_type=jnp.float32)
            x = jnp.maximum(z + x, 0.0)

        logits = eb_ref[:, :] + jnp.dot(x, ew_ref[:, :],
                                        preferred_element_type=jnp.float32)
        m = jnp.max(logits)
        e = jnp.exp(logits - m)
        p = e / jnp.sum(e)
        cum = jnp.dot(p, tri_ref[:, :], preferred_element_type=jnp.float32,
                      precision=jax.lax.Precision.HIGHEST)
        s = s_ref[pl.ds(t, 1), :]
        cnt = jnp.sum((cum <= s).astype(jnp.int32))
        nx = jnp.bitwise_and(cnt, NUM_CLS - 1)
        out_ref[pl.ds(t, 1), :] = nx[None, None]
        idxh[t + DILATIONS[0] + 1, 0] = nx
        return nx

    jax.lax.fori_loop(0, T, step, jnp.int32(NUM_CLS // 2 - 1))


def kernel(y, samples, emb, condition_W, WV_past_weight, WV_present_weight,
           W_o_weight, W_o_bias, end_w, end_b):
    y_up_t = jnp.repeat(y, HOP, axis=1).T
    cw_t = condition_W.T
    wp_t = WV_past_weight[:, :, :, 0].transpose(0, 2, 1)
    wpr_t = WV_present_weight.transpose(0, 2, 1)
    wo_t = W_o_weight.transpose(0, 2, 1)
    ew_t = end_w.T
    s2 = samples.reshape(T, 1)
    eb2 = end_b.reshape(1, NUM_CLS)
    tri = (jnp.arange(NUM_CLS)[:, None] <= jnp.arange(NUM_CLS)[None, :]
           ).astype(jnp.float32)

    hist_scratch = [pltpu.VMEM((T + DILATIONS[j], FFT), jnp.float32)
                    for j in range(1, L)]
    out = pl.pallas_call(
        _body,
        out_shape=jax.ShapeDtypeStruct((T, 1), jnp.int32),
        scratch_shapes=hist_scratch + [
            pltpu.VMEM((T, L * FFT), jnp.float32),
            pltpu.VMEM((NUM_CLS + 8, FFT), jnp.float32),
            pltpu.VMEM((NUM_CLS, FFT), jnp.float32),
            pltpu.SMEM((T + DILATIONS[0] + 1, 1), jnp.int32),
        ],
    )(y_up_t, s2, emb, cw_t, wp_t, wpr_t, wo_t, W_o_bias, ew_t, eb2, tri)
    return out[:, 0]


# speedup vs baseline: 1.1225x; 1.0233x over previous
import jax
import jax.numpy as jnp
from jax.experimental import pallas as pl
from jax.experimental.pallas import tpu as pltpu

L = 8


FFT = 256
NUM_CLS = 256
COND = 80
HOP = 64
FRAMES = 4
T = FRAMES * HOP
DILATIONS = [128, 64, 32, 16, 8, 4, 2, 1]


def _body(yup_ref, s_ref, emb_ref, cw_ref, wp_ref, wpr_ref, wo_ref, wob_ref,
          ew_ref, eb_ref, tri_ref, out_ref,
          h1, h2, h3, h4, h5, h6, h7, conds, m0p, m0r, idxh):
    conds[:, :] = jnp.dot(yup_ref[:, :], cw_ref[:, :],
                          preferred_element_type=jnp.float32)
    m0p[0:NUM_CLS, :] = jnp.dot(emb_ref[:, :], wp_ref[0],
                                preferred_element_type=jnp.float32)
    m0p[NUM_CLS:NUM_CLS + 8, :] = jnp.zeros((8, FFT), jnp.float32)
    m0r[:, :] = jnp.dot(emb_ref[:, :], wpr_ref[0],
                        preferred_element_type=jnp.float32)
    hists = [h1, h2, h3, h4, h5, h6, h7]
    for j in range(1, L):
        d = DILATIONS[j]
        hists[j - 1][0:d, :] = jnp.zeros((d, FFT), jnp.float32)

    def init_idx(u, c):
        idxh[u, 0] = NUM_CLS
        return c
    jax.lax.fori_loop(0, DILATIONS[0], init_idx, 0)
    idxh[DILATIONS[0], 0] = NUM_CLS // 2 - 1

    def step(t, k):
        kp = idxh[t, 0]
        h = conds[pl.ds(t, 1), 0:FFT] + m0p[pl.ds(kp, 1), :]
        h = h + m0r[pl.ds(k, 1), :]
        h = jnp.maximum(h, 0.0)
        z = wob_ref[0:1, :] + jnp.dot(h, wo_ref[0],
                                      preferred_element_type=jnp.float32)
        x = jnp.maximum(z + emb_ref[pl.ds(k, 1), :], 0.0)

        for j in range(1, L):
            d = DILATIONS[j]
            hj = hists[j - 1]
            xpast = hj[pl.ds(t, 1), :]
            h = conds[pl.ds(t, 1), j * FFT:(j + 1) * FFT]
            h = h + jnp.dot(xpast, wp_ref[j], preferred_element_type=jnp.float32)
            h = h + jnp.dot(x, wpr_ref[j], preferred_element_type=jnp.float32)
            hj[pl.ds(t + d, 1), :] = x
            h = jnp.maximum(h, 0.0)
            z = wob_ref[j:j + 1, :] + jnp.dot(h, wo_ref[j],
                                              preferred_element_type=jnp.float32)
            x = jnp.maximum(z + x, 0.0)

        logits = eb_ref[:, :] + jnp.dot(x, ew_ref[:, :],
                                        preferred_element_type=jnp.float32)
        m = jnp.max(logits)
        e = jnp.exp(logits - m)
        cum_e = jnp.dot(e, tri_ref[:, :], preferred_element_type=jnp.float32,
                        precision=jax.lax.Precision.HIGHEST)
        cum = cum_e / jnp.sum(e)
        s = s_ref[t, 0]
        nx = jnp.argmax((cum > s).astype(jnp.float32)).astype(jnp.int32)
        out_ref[pl.ds(t, 1), :] = jnp.full((1, 1), nx, jnp.int32)
        idxh[t + DILATIONS[0] + 1, 0] = nx
        return nx

    jax.lax.fori_loop(0, T, step, jnp.int32(NUM_CLS // 2 - 1))


def kernel(y, samples, emb, condition_W, WV_past_weight, WV_present_weight,
           W_o_weight, W_o_bias, end_w, end_b):
    y_up_t = jnp.repeat(y, HOP, axis=1).T
    cw_t = condition_W.T
    wp_t = WV_past_weight[:, :, :, 0].transpose(0, 2, 1)
    wpr_t = WV_present_weight.transpose(0, 2, 1)
    wo_t = W_o_weight.transpose(0, 2, 1)
    ew_t = end_w.T
    s2 = samples.reshape(T, 1)
    eb2 = end_b.reshape(1, NUM_CLS)
    tri = (jnp.arange(NUM_CLS)[:, None] <= jnp.arange(NUM_CLS)[None, :]
           ).astype(jnp.float32)

    hist_scratch = [pltpu.VMEM((T + DILATIONS[j], FFT), jnp.float32)
                    for j in range(1, L)]
    vmem_spec = pl.BlockSpec(memory_space=pltpu.VMEM)
    out = pl.pallas_call(
        _body,
        out_shape=jax.ShapeDtypeStruct((T, 1), jnp.int32),
        in_specs=[vmem_spec, pl.BlockSpec(memory_space=pltpu.SMEM)] +
                 [vmem_spec] * 9,
        scratch_shapes=hist_scratch + [
            pltpu.VMEM((T, L * FFT), jnp.float32),
            pltpu.VMEM((NUM_CLS + 8, FFT), jnp.float32),
            pltpu.VMEM((NUM_CLS, FFT), jnp.float32),
            pltpu.SMEM((T + DILATIONS[0] + 1, 1), jnp.int32),
        ],
    )(y_up_t, s2, emb, cw_t, wp_t, wpr_t, wo_t, W_o_bias, ew_t, eb2, tri)
    return out[:, 0]


# speedup vs baseline: 1.1430x; 1.0183x over previous
import jax
import jax.numpy as jnp
from jax.experimental import pallas as pl
from jax.experimental.pallas import tpu as pltpu

L = 8


FFT = 256
NUM_CLS = 256
COND = 80
HOP = 64
FRAMES = 4
T = FRAMES * HOP
DILATIONS = [128, 64, 32, 16, 8, 4, 2, 1]


def _body(yup_ref, s_ref, emb_ref, cw_ref, wp_ref, wpr_ref, wo_ref, wob_ref,
          ew_ref, eb_ref, tri_ref, out_ref,
          h1, h2, h3, h4, h5, h6, h7, conds, m0p, m0r, idxh):
    conds[:, :] = jnp.dot(yup_ref[:, :], cw_ref[:, :],
                          preferred_element_type=jnp.float32)
    m0p[0:NUM_CLS, :] = jnp.dot(emb_ref[:, :], wp_ref[0],
                                preferred_element_type=jnp.float32)
    m0p[NUM_CLS:NUM_CLS + 8, :] = jnp.zeros((8, FFT), jnp.float32)
    m0r[:, :] = jnp.dot(emb_ref[:, :], wpr_ref[0],
                        preferred_element_type=jnp.float32)
    hists = [h1, h2, h3, h4, h5, h6, h7]
    for j in range(1, L):
        d = DILATIONS[j]
        hists[j - 1][0:d, :] = jnp.zeros((d, FFT), jnp.float32)

    def init_idx(u, c):
        idxh[u, 0] = NUM_CLS
        return c
    jax.lax.fori_loop(0, DILATIONS[0], init_idx, 0)
    idxh[DILATIONS[0], 0] = NUM_CLS // 2 - 1

    def step(t, k):
        kp = idxh[t, 0]
        h = conds[pl.ds(t, 1), 0:FFT] + m0p[pl.ds(kp, 1), :]
        h = h + m0r[pl.ds(k, 1), :]
        h = jnp.maximum(h, 0.0)
        z = wob_ref[0:1, :] + jnp.dot(h, wo_ref[0],
                                      preferred_element_type=jnp.float32)
        x = jnp.maximum(z + emb_ref[pl.ds(k, 1), :], 0.0)

        for j in range(1, L):
            d = DILATIONS[j]
            hj = hists[j - 1]
            xpast = hj[pl.ds(t, 1), :]
            h = conds[pl.ds(t, 1), j * FFT:(j + 1) * FFT]
            h = h + jnp.dot(xpast, wp_ref[j], preferred_element_type=jnp.float32)
            h = h + jnp.dot(x, wpr_ref[j], preferred_element_type=jnp.float32)
            hj[pl.ds(t + d, 1), :] = x
            h = jnp.maximum(h, 0.0)
            z = wob_ref[j:j + 1, :] + jnp.dot(h, wo_ref[j],
                                              preferred_element_type=jnp.float32)
            x = jnp.maximum(z + x, 0.0)

        logits = eb_ref[:, :] + jnp.dot(x, ew_ref[:, :],
                                        preferred_element_type=jnp.float32)
        m = jnp.max(logits)
        e = jnp.exp(logits - m)
        cum_e = jnp.dot(e, tri_ref[:, :], preferred_element_type=jnp.float32,
                        precision=jax.lax.Precision.HIGHEST)
        cum = cum_e / jnp.sum(e)
        s = s_ref[t, 0]
        cnt = jnp.sum((cum <= s).astype(jnp.int32))
        nx = jnp.bitwise_and(cnt, NUM_CLS - 1)
        out_ref[pl.ds(t, 1), :] = jnp.full((1, 1), nx, jnp.int32)
        idxh[t + DILATIONS[0] + 1, 0] = nx
        return nx

    jax.lax.fori_loop(0, T, step, jnp.int32(NUM_CLS // 2 - 1))


def kernel(y, samples, emb, condition_W, WV_past_weight, WV_present_weight,
           W_o_weight, W_o_bias, end_w, end_b):
    y_up_t = jnp.repeat(y, HOP, axis=1).T
    cw_t = condition_W.T
    wp_t = WV_past_weight[:, :, :, 0].transpose(0, 2, 1)
    wpr_t = WV_present_weight.transpose(0, 2, 1)
    wo_t = W_o_weight.transpose(0, 2, 1)
    ew_t = end_w.T
    s2 = samples.reshape(T, 1)
    eb2 = end_b.reshape(1, NUM_CLS)
    tri = (jnp.arange(NUM_CLS)[:, None] <= jnp.arange(NUM_CLS)[None, :]
           ).astype(jnp.float32)

    hist_scratch = [pltpu.VMEM((T + DILATIONS[j], FFT), jnp.float32)
                    for j in range(1, L)]
    vmem_spec = pl.BlockSpec(memory_space=pltpu.VMEM)
    out = pl.pallas_call(
        _body,
        out_shape=jax.ShapeDtypeStruct((T, 1), jnp.int32),
        in_specs=[vmem_spec, pl.BlockSpec(memory_space=pltpu.SMEM)] +
                 [vmem_spec] * 9,
        scratch_shapes=hist_scratch + [
            pltpu.VMEM((T, L * FFT), jnp.float32),
            pltpu.VMEM((NUM_CLS + 8, FFT), jnp.float32),
            pltpu.VMEM((NUM_CLS, FFT), jnp.float32),
            pltpu.SMEM((T + DILATIONS[0] + 1, 1), jnp.int32),
        ],
    )(y_up_t, s2, emb, cw_t, wp_t, wpr_t, wo_t, W_o_bias, ew_t, eb2, tri)
    return out[:, 0]
